# Initial kernel scaffold; baseline (speedup 1.0000x reference)
import jax
import jax.numpy as jnp
from jax.experimental import pallas as pl


def kernel(data):
    raise NotImplementedError("write your pallas kernel here")


# speedup vs baseline: 3.2178x; 3.2178x over previous
import jax
import jax.numpy as jnp
from jax.experimental import pallas as pl
from jax.experimental.pallas import tpu as pltpu

_B = 4
_N = 16384
_D = 128
_C = 64


def _kmpp_body(init_ref, data_ref, dataT_ref, out_ref, mind2_ref, x2_ref):
    b = pl.program_id(0)
    dt = dataT_ref[0]
    x2_ref[:, :] = jnp.sum(dt * dt, axis=0, keepdims=True)
    i0 = init_ref[b]
    out_ref[0, 0:1, :] = data_ref[0, pl.ds(i0, 1), :]
    mind2_ref[:, :] = jnp.full((1, _N), jnp.inf, dtype=jnp.float32)

    iota = jax.lax.broadcasted_iota(jnp.int32, (1, _N), 1)

    def step(i, carry):
        cprev = out_ref[0, pl.ds(i - 1, 1), :]
        dot = jax.lax.dot_general(
            cprev,
            dataT_ref[0],
            (((1,), (0,)), ((), ())),
            preferred_element_type=jnp.float32,
        )
        c2 = jnp.sum(cprev * cprev)
        d2 = jnp.maximum(x2_ref[:, :] + c2 - 2.0 * dot, 0.0)
        m = jnp.minimum(mind2_ref[:, :], d2)
        mind2_ref[:, :] = m
        mx = jnp.max(m)
        idx = jnp.min(jnp.where(m == mx, iota, _N))
        out_ref[0, pl.ds(i, 1), :] = data_ref[0, pl.ds(idx, 1), :]
        return carry

    jax.lax.fori_loop(1, _C, step, 0)


def kernel(data):
    b, n, d = data.shape
    init_key = jax.random.key(42)
    init_idx = jax.random.randint(init_key, (b,), 0, n).astype(jnp.int32)
    dataT = jnp.swapaxes(data, 1, 2)
    return pl.pallas_call(
        _kmpp_body,
        grid=(b,),
        out_shape=jax.ShapeDtypeStruct((b, _C, d), jnp.float32),
        in_specs=[
            pl.BlockSpec(memory_space=pltpu.SMEM),
            pl.BlockSpec((1, n, d), lambda i: (i, 0, 0)),
            pl.BlockSpec((1, d, n), lambda i: (i, 0, 0)),
        ],
        out_specs=pl.BlockSpec((1, _C, d), lambda i: (i, 0, 0)),
        scratch_shapes=[
            pltpu.VMEM((1, _N), jnp.float32),
            pltpu.VMEM((1, _N), jnp.float32),
        ],
        compiler_params=pltpu.CompilerParams(
            vmem_limit_bytes=100 * 1024 * 1024,
        ),
    )(init_idx, data, dataT)
